# Initial kernel scaffold; baseline (speedup 1.0000x reference)
import jax
import jax.numpy as jnp
from jax.experimental import pallas as pl


def kernel(x, edge_index, edge_attr, enc_nW1, enc_nb1, enc_nW2, enc_nb2, enc_eW1, enc_eb1, enc_eW2, enc_eb2, relW1, relb1, relW2, relb2, objW1, objb1, objW2, objb2, betaW1, betab1, betaW2, betab2, hW1, hb1, hW2, hb2):
    raise NotImplementedError("write your pallas kernel here")


# speedup vs baseline: 2.4253x; 2.4253x over previous
import functools

import jax
import jax.numpy as jnp
from jax import lax
from jax.experimental import pallas as pl
from jax.experimental.pallas import tpu as pltpu
from jax.experimental.pallas import tpu_sc as plsc

ALPHA = 0.5
N_NODES = 10000
N_EDGES = 320000
NPAD = 10016
EPAD = 323584
NW = 32
CH = 128
NCH_S = 79
NCH_G = 158
TW = 8
BE = 2048
STRIPE = NPAD // 16


def _sc_gather(table, idx):
    mesh = plsc.VectorSubcoreMesh(core_axis_name="c", subcore_axis_name="s")

    @functools.partial(
        pl.kernel,
        mesh=mesh,
        out_type=jax.ShapeDtypeStruct((NW, NCH_G, CH, TW), jnp.float32),
        compiler_params=pltpu.CompilerParams(use_tc_tiling_on_sc=False),
        scratch_types=[
            pltpu.VMEM((NCH_G, CH), jnp.int32),
            pltpu.VMEM((CH, TW), jnp.float32),
            pltpu.SemaphoreType.DMA,
        ],
    )
    def k(table_hbm, idx_hbm, out_hbm, idx_v, rows_v, sem):
        w = lax.axis_index("s") * 2 + lax.axis_index("c")
        pltpu.sync_copy(idx_hbm.at[w], idx_v)

        def body(j, carry):
            pltpu.async_copy(table_hbm.at[idx_v.at[j]], rows_v, sem).wait()
            pltpu.sync_copy(rows_v, out_hbm.at[w, j])
            return carry

        lax.fori_loop(0, NCH_G, body, 0)

    return k(table, idx)


def _sc_scatter_add(vals, idx, zeros_tab):
    mesh = plsc.VectorSubcoreMesh(core_axis_name="c", subcore_axis_name="s")

    @functools.partial(
        pl.kernel,
        mesh=mesh,
        out_type=jax.ShapeDtypeStruct((2, NPAD, TW), jnp.float32),
        compiler_params=pltpu.CompilerParams(use_tc_tiling_on_sc=False),
        scratch_types=[
            pltpu.VMEM((NCH_S, CH), jnp.int32),
            pltpu.VMEM((CH, TW), jnp.float32),
            pltpu.VMEM((STRIPE, TW), jnp.float32),
            pltpu.VMEM_SHARED((NPAD, TW), jnp.float32),
            pltpu.SemaphoreType.DMA,
        ],
    )
    def k(vals_hbm, idx_hbm, z_hbm, out_hbm, idx_v, vals_v, stripe_v, acc_sh, sem):
        c = lax.axis_index("c")
        s = lax.axis_index("s")
        w = s * 2 + c
        pltpu.sync_copy(z_hbm.at[pl.ds(s * STRIPE, STRIPE)], stripe_v)
        pltpu.sync_copy(stripe_v, acc_sh.at[pl.ds(s * STRIPE, STRIPE)])
        plsc.subcore_barrier()
        pltpu.sync_copy(idx_hbm.at[w], idx_v)

        def body(j, carry):
            pltpu.sync_copy(vals_hbm.at[w, j], vals_v)
            pltpu.sync_copy(vals_v, acc_sh.at[idx_v.at[j]], add=True)
            return carry

        lax.fori_loop(0, NCH_S, body, 0)
        plsc.subcore_barrier()
        pltpu.sync_copy(acc_sh.at[pl.ds(s * STRIPE, STRIPE)], stripe_v)
        pltpu.sync_copy(stripe_v, out_hbm.at[c, pl.ds(s * STRIPE, STRIPE)])

    return k(vals, idx, zeros_tab)


def _tc_node_encoder(xp, W1, b1, W2p, b2p):
    def body(x_ref, W1_ref, b1_ref, W2_ref, b2_ref, o_ref):
        hid = jnp.dot(x_ref[...], W1_ref[...], preferred_element_type=jnp.float32)
        hid = jnp.maximum(hid + b1_ref[...], 0.0)
        o_ref[...] = jnp.dot(hid, W2_ref[...], preferred_element_type=jnp.float32) + b2_ref[...]

    return pl.pallas_call(
        body,
        out_shape=jax.ShapeDtypeStruct((NPAD, TW), jnp.float32),
    )(xp, W1, b1, W2p, b2p)


def _tc_edge_encoder(eap, W1, b1, W2p, b2p):
    nb = EPAD // BE

    def body(a_ref, W1_ref, b1_ref, W2_ref, b2_ref, o_ref):
        hid = jnp.dot(a_ref[...], W1_ref[...], preferred_element_type=jnp.float32)
        hid = jnp.maximum(hid + b1_ref[...], 0.0)
        o_ref[...] = jnp.dot(hid, W2_ref[...], preferred_element_type=jnp.float32) + b2_ref[...]

    return pl.pallas_call(
        body,
        grid=(nb,),
        in_specs=[
            pl.BlockSpec((BE, 16), lambda i: (i, 0)),
            pl.BlockSpec((16, 40), lambda i: (0, 0)),
            pl.BlockSpec((1, 40), lambda i: (0, 0)),
            pl.BlockSpec((40, TW), lambda i: (0, 0)),
            pl.BlockSpec((1, TW), lambda i: (0, 0)),
        ],
        out_specs=pl.BlockSpec((BE, TW), lambda i: (i, 0)),
        out_shape=jax.ShapeDtypeStruct((EPAD, TW), jnp.float32),
        compiler_params=pltpu.CompilerParams(dimension_semantics=("parallel",)),
    )(eap, W1, b1, W2p, b2p)


def _tc_edge_layer(hflat, e, W1s, W1d, W1e, b1, W2p, b2p, want_next):
    nb = EPAD // BE

    def body(hs_ref, hd_ref, e_ref, W1s_ref, W1d_ref, W1e_ref, b1_ref, W2_ref,
             b2_ref, en_ref, *rest):
        hid = jnp.dot(hs_ref[...], W1s_ref[...], preferred_element_type=jnp.float32)
        hid = hid + jnp.dot(hd_ref[...], W1d_ref[...], preferred_element_type=jnp.float32)
        hid = hid + jnp.dot(e_ref[...], W1e_ref[...], preferred_element_type=jnp.float32)
        hid = jnp.maximum(hid + b1_ref[...], 0.0)
        en = jnp.dot(hid, W2_ref[...], preferred_element_type=jnp.float32) + b2_ref[...]
        en_ref[...] = en
        if rest:
            rest[0][...] = ALPHA * e_ref[...] + (1.0 - ALPHA) * en

    out_shapes = [jax.ShapeDtypeStruct((EPAD, TW), jnp.float32)]
    out_specs = [pl.BlockSpec((BE, TW), lambda i: (i, 0))]
    if want_next:
        out_shapes.append(jax.ShapeDtypeStruct((EPAD, TW), jnp.float32))
        out_specs.append(pl.BlockSpec((BE, TW), lambda i: (i, 0)))

    return pl.pallas_call(
        body,
        grid=(nb,),
        in_specs=[
            pl.BlockSpec((BE, TW), lambda i: (i, 0)),
            pl.BlockSpec((BE, TW), lambda i: (i + nb, 0)),
            pl.BlockSpec((BE, TW), lambda i: (i, 0)),
            pl.BlockSpec((TW, 40), lambda i: (0, 0)),
            pl.BlockSpec((TW, 40), lambda i: (0, 0)),
            pl.BlockSpec((TW, 40), lambda i: (0, 0)),
            pl.BlockSpec((1, 40), lambda i: (0, 0)),
            pl.BlockSpec((40, TW), lambda i: (0, 0)),
            pl.BlockSpec((1, TW), lambda i: (0, 0)),
        ],
        out_specs=out_specs,
        out_shape=out_shapes,
        compiler_params=pltpu.CompilerParams(dimension_semantics=("parallel",)),
    )(hflat, hflat, e, W1s, W1d, W1e, b1, W2p, b2p)


def _tc_node_layer(htab, parts, Wh, Wa, b1, W2p, b2p):
    def body(h_ref, p_ref, Wh_ref, Wa_ref, b1_ref, W2_ref, b2_ref, o_ref):
        h = h_ref[...]
        agg = p_ref[0] + p_ref[1]
        hid = jnp.dot(h, Wh_ref[...], preferred_element_type=jnp.float32)
        hid = hid + jnp.dot(agg, Wa_ref[...], preferred_element_type=jnp.float32)
        hid = jnp.maximum(hid + b1_ref[...], 0.0)
        hn = jnp.dot(hid, W2_ref[...], preferred_element_type=jnp.float32) + b2_ref[...]
        o_ref[...] = ALPHA * h + (1.0 - ALPHA) * hn

    return pl.pallas_call(
        body,
        out_shape=jax.ShapeDtypeStruct((NPAD, TW), jnp.float32),
    )(htab, parts, Wh, Wa, b1, W2p, b2p)


def _tc_final(htab, parts, Wh, Wa, b1, W2p, b2p, Wb1, bb1, Wb2, bb2, Wh1, bh1, Wh2, bh2):
    def body(h_ref, p_ref, Wh_ref, Wa_ref, b1_ref, W2_ref, b2_ref,
             Wb1_ref, bb1_ref, Wb2_ref, bb2_ref, Wh1_ref, bh1_ref, Wh2_ref,
             bh2_ref, o_ref):
        h = h_ref[...]
        agg = p_ref[0] + p_ref[1]
        hid = jnp.dot(h, Wh_ref[...], preferred_element_type=jnp.float32)
        hid = hid + jnp.dot(agg, Wa_ref[...], preferred_element_type=jnp.float32)
        hid = jnp.maximum(hid + b1_ref[...], 0.0)
        hn = jnp.dot(hid, W2_ref[...], preferred_element_type=jnp.float32) + b2_ref[...]
        hf = (ALPHA * h + (1.0 - ALPHA) * hn)[:N_NODES]
        bhid = jnp.maximum(
            jnp.dot(hf, Wb1_ref[...], preferred_element_type=jnp.float32) + bb1_ref[...], 0.0)
        b8 = jnp.dot(bhid, Wb2_ref[...], preferred_element_type=jnp.float32) + bb2_ref[...]
        hhid = jnp.maximum(
            jnp.dot(hf, Wh1_ref[...], preferred_element_type=jnp.float32) + bh1_ref[...], 0.0)
        h8 = jnp.dot(hhid, Wh2_ref[...], preferred_element_type=jnp.float32) + bh2_ref[...]
        beta = jax.nn.sigmoid(b8[:, 0:1])
        o_ref[...] = jnp.concatenate([h8[:, 0:2], beta], axis=1)

    return pl.pallas_call(
        body,
        out_shape=jax.ShapeDtypeStruct((N_NODES, 3), jnp.float32),
    )(htab, parts, Wh, Wa, b1, W2p, b2p, Wb1, bb1, Wb2, bb2, Wh1, bh1, Wh2, bh2)


def _pad_rows(W, rows):
    out = jnp.zeros((rows, W.shape[1]), W.dtype)
    return out.at[: W.shape[0]].set(W)


def _pad_cols(W, cols):
    out = jnp.zeros((W.shape[0], cols), W.dtype)
    return out.at[:, : W.shape[1]].set(W)


def kernel(x, edge_index, edge_attr, enc_nW1, enc_nb1, enc_nW2, enc_nb2,
           enc_eW1, enc_eb1, enc_eW2, enc_eb2, relW1, relb1, relW2, relb2,
           objW1, objb1, objW2, objb2, betaW1, betab1, betaW2, betab2,
           hW1, hb1, hW2, hb2):
    f32 = jnp.float32
    i32 = jnp.int32
    src = edge_index[0]
    dst = edge_index[1]
    padi = jnp.full((EPAD - N_EDGES,), N_NODES, i32)
    src_p = jnp.concatenate([src, padi])
    dst_p = jnp.concatenate([dst, padi])
    gidx = jnp.concatenate([src_p, dst_p]).reshape(NW, NCH_G, CH)
    sidx = dst_p.reshape(NW, NCH_S, CH)
    xp = jnp.pad(x, ((0, NPAD - N_NODES), (0, 0)))
    eap = jnp.pad(edge_attr, ((0, EPAD - N_EDGES), (0, 0)))
    zeros_tab = jnp.zeros((NPAD, TW), f32)

    htab = _tc_node_encoder(
        xp, enc_nW1, enc_nb1.reshape(1, 40), _pad_cols(enc_nW2, TW),
        _pad_cols(enc_nb2.reshape(1, 5), TW))
    e = _tc_edge_encoder(
        eap, enc_eW1, enc_eb1.reshape(1, 40), _pad_cols(enc_eW2, TW),
        _pad_cols(enc_eb2.reshape(1, 4), TW))

    out = None
    for l in range(relW1.shape[0]):
        W1 = relW1[l]
        W1s = _pad_rows(W1[0:5], TW)
        W1d = _pad_rows(W1[5:10], TW)
        W1e = _pad_rows(W1[10:14], TW)
        b1 = relb1[l].reshape(1, 40)
        W2p = _pad_cols(relW2[l], TW)
        b2p = _pad_cols(relb2[l].reshape(1, 4), TW)

        gout = _sc_gather(htab, gidx)
        hflat = gout.reshape(2 * EPAD, TW)
        last = l == relW1.shape[0] - 1
        if last:
            (e_new,) = _tc_edge_layer(hflat, e, W1s, W1d, W1e, b1, W2p, b2p, False)
        else:
            e_new, e = _tc_edge_layer(hflat, e, W1s, W1d, W1e, b1, W2p, b2p, True)

        parts = _sc_scatter_add(e_new.reshape(NW, NCH_S, CH, TW), sidx, zeros_tab)

        oW1 = objW1[l]
        Wh = _pad_rows(oW1[0:5], TW)
        Wa = _pad_rows(oW1[5:9], TW)
        ob1 = objb1[l].reshape(1, 40)
        oW2p = _pad_cols(objW2[l], TW)
        ob2p = _pad_cols(objb2[l].reshape(1, 5), TW)
        if not last:
            htab = _tc_node_layer(htab, parts, Wh, Wa, ob1, oW2p, ob2p)
        else:
            out = _tc_final(
                htab, parts, Wh, Wa, ob1, oW2p, ob2p,
                _pad_rows(betaW1, TW), betab1.reshape(1, 40),
                _pad_cols(betaW2, TW), _pad_cols(betab2.reshape(1, 1), TW),
                _pad_rows(hW1, TW), hb1.reshape(1, 40),
                _pad_cols(hW2, TW), _pad_cols(hb2.reshape(1, 2), TW))
    return out


# speedup vs baseline: 2.6992x; 1.1129x over previous
import functools

import jax
import jax.numpy as jnp
from jax import lax
from jax.experimental import pallas as pl
from jax.experimental.pallas import tpu as pltpu
from jax.experimental.pallas import tpu_sc as plsc

ALPHA = 0.5
N_NODES = 10000
N_EDGES = 320000
NPAD = 10016
EPAD = 323584
NW = 32
CH = 128
NCH_S = 79
NCH_G = 158
TW = 8
BE = 2048
STRIPE = NPAD // 16


NBUF = 12
PREF = 6


def _sc_gather(table, idx):
    mesh = plsc.VectorSubcoreMesh(core_axis_name="c", subcore_axis_name="s")

    @functools.partial(
        pl.kernel,
        mesh=mesh,
        out_type=jax.ShapeDtypeStruct((NW, NCH_G, CH, TW), jnp.float32),
        compiler_params=pltpu.CompilerParams(use_tc_tiling_on_sc=False),
        scratch_types=[
            pltpu.VMEM((NCH_G, CH), jnp.int32),
            pltpu.VMEM((NBUF, CH, TW), jnp.float32),
            pltpu.SemaphoreType.DMA((NBUF,)),
            pltpu.SemaphoreType.DMA((NBUF,)),
        ],
    )
    def k(table_hbm, idx_hbm, out_hbm, idx_v, rows_v, gsem, osem):
        w = lax.axis_index("s") * 2 + lax.axis_index("c")
        pltpu.sync_copy(idx_hbm.at[w], idx_v)
        for b in range(PREF):
            pltpu.async_copy(table_hbm.at[idx_v.at[b]], rows_v.at[b], gsem.at[b])

        def body(j, carry):
            jn = j + PREF
            bn = jn % NBUF
            b = j % NBUF

            @pl.when(jn < NCH_G)
            def _():
                @pl.when(jn >= NBUF)
                def _():
                    pltpu.make_async_copy(rows_v.at[bn], out_hbm.at[w, 0],
                                          osem.at[bn]).wait()
                pltpu.async_copy(table_hbm.at[idx_v.at[jn]], rows_v.at[bn],
                                 gsem.at[bn])

            pltpu.make_async_copy(table_hbm.at[pl.ds(0, CH)], rows_v.at[b],
                                  gsem.at[b]).wait()
            pltpu.async_copy(rows_v.at[b], out_hbm.at[w, j], osem.at[b])
            return carry

        lax.fori_loop(0, NCH_G, body, 0)
        for b in range(NBUF):
            pltpu.make_async_copy(rows_v.at[b], out_hbm.at[w, 0],
                                  osem.at[b]).wait()

    return k(table, idx)


def _sc_scatter_add(vals, idx, zeros_tab):
    mesh = plsc.VectorSubcoreMesh(core_axis_name="c", subcore_axis_name="s")

    @functools.partial(
        pl.kernel,
        mesh=mesh,
        out_type=jax.ShapeDtypeStruct((2, NPAD, TW), jnp.float32),
        compiler_params=pltpu.CompilerParams(use_tc_tiling_on_sc=False),
        scratch_types=[
            pltpu.VMEM((NCH_S, CH), jnp.int32),
            pltpu.VMEM((NBUF, CH, TW), jnp.float32),
            pltpu.VMEM((STRIPE, TW), jnp.float32),
            pltpu.VMEM_SHARED((NPAD, TW), jnp.float32),
            pltpu.SemaphoreType.DMA((NBUF,)),
            pltpu.SemaphoreType.DMA((NBUF,)),
        ],
    )
    def k(vals_hbm, idx_hbm, z_hbm, out_hbm, idx_v, vals_v, stripe_v, acc_sh,
          vsem, ssem):
        c = lax.axis_index("c")
        s = lax.axis_index("s")
        w = s * 2 + c
        pltpu.sync_copy(z_hbm.at[pl.ds(s * STRIPE, STRIPE)], stripe_v)
        pltpu.sync_copy(stripe_v, acc_sh.at[pl.ds(s * STRIPE, STRIPE)])
        plsc.subcore_barrier()
        pltpu.sync_copy(idx_hbm.at[w], idx_v)
        for b in range(PREF):
            pltpu.async_copy(vals_hbm.at[w, b], vals_v.at[b], vsem.at[b])

        def body(j, carry):
            jn = j + PREF
            bn = jn % NBUF
            b = j % NBUF

            @pl.when(jn < NCH_S)
            def _():
                @pl.when(jn >= NBUF)
                def _():
                    pltpu.make_async_copy(vals_v.at[bn],
                                          acc_sh.at[pl.ds(0, CH)],
                                          ssem.at[bn]).wait()
                pltpu.async_copy(vals_hbm.at[w, jn], vals_v.at[bn], vsem.at[bn])

            pltpu.make_async_copy(vals_hbm.at[w, 0], vals_v.at[b],
                                  vsem.at[b]).wait()
            pltpu.async_copy(vals_v.at[b], acc_sh.at[idx_v.at[j]], ssem.at[b],
                             add=True)
            return carry

        lax.fori_loop(0, NCH_S, body, 0)
        for b in range(NBUF):
            pltpu.make_async_copy(vals_v.at[b], acc_sh.at[pl.ds(0, CH)],
                                  ssem.at[b]).wait()
        plsc.subcore_barrier()
        pltpu.sync_copy(acc_sh.at[pl.ds(s * STRIPE, STRIPE)], stripe_v)
        pltpu.sync_copy(stripe_v, out_hbm.at[c, pl.ds(s * STRIPE, STRIPE)])

    return k(vals, idx, zeros_tab)


def _tc_node_encoder(xp, W1, b1, W2p, b2p):
    def body(x_ref, W1_ref, b1_ref, W2_ref, b2_ref, o_ref):
        hid = jnp.dot(x_ref[...], W1_ref[...], preferred_element_type=jnp.float32)
        hid = jnp.maximum(hid + b1_ref[...], 0.0)
        o_ref[...] = jnp.dot(hid, W2_ref[...], preferred_element_type=jnp.float32) + b2_ref[...]

    return pl.pallas_call(
        body,
        out_shape=jax.ShapeDtypeStruct((NPAD, TW), jnp.float32),
    )(xp, W1, b1, W2p, b2p)


def _tc_edge_encoder(eap, W1, b1, W2p, b2p):
    nb = EPAD // BE

    def body(a_ref, W1_ref, b1_ref, W2_ref, b2_ref, o_ref):
        hid = jnp.dot(a_ref[...], W1_ref[...], preferred_element_type=jnp.float32)
        hid = jnp.maximum(hid + b1_ref[...], 0.0)
        o_ref[...] = jnp.dot(hid, W2_ref[...], preferred_element_type=jnp.float32) + b2_ref[...]

    return pl.pallas_call(
        body,
        grid=(nb,),
        in_specs=[
            pl.BlockSpec((BE, 16), lambda i: (i, 0)),
            pl.BlockSpec((16, 40), lambda i: (0, 0)),
            pl.BlockSpec((1, 40), lambda i: (0, 0)),
            pl.BlockSpec((40, TW), lambda i: (0, 0)),
            pl.BlockSpec((1, TW), lambda i: (0, 0)),
        ],
        out_specs=pl.BlockSpec((BE, TW), lambda i: (i, 0)),
        out_shape=jax.ShapeDtypeStruct((EPAD, TW), jnp.float32),
        compiler_params=pltpu.CompilerParams(dimension_semantics=("parallel",)),
    )(eap, W1, b1, W2p, b2p)


def _tc_edge_layer(hflat, e, W1s, W1d, W1e, b1, W2p, b2p, want_next):
    nb = EPAD // BE

    def body(hs_ref, hd_ref, e_ref, W1s_ref, W1d_ref, W1e_ref, b1_ref, W2_ref,
             b2_ref, en_ref, *rest):
        hid = jnp.dot(hs_ref[...], W1s_ref[...], preferred_element_type=jnp.float32)
        hid = hid + jnp.dot(hd_ref[...], W1d_ref[...], preferred_element_type=jnp.float32)
        hid = hid + jnp.dot(e_ref[...], W1e_ref[...], preferred_element_type=jnp.float32)
        hid = jnp.maximum(hid + b1_ref[...], 0.0)
        en = jnp.dot(hid, W2_ref[...], preferred_element_type=jnp.float32) + b2_ref[...]
        en_ref[...] = en
        if rest:
            rest[0][...] = ALPHA * e_ref[...] + (1.0 - ALPHA) * en

    out_shapes = [jax.ShapeDtypeStruct((EPAD, TW), jnp.float32)]
    out_specs = [pl.BlockSpec((BE, TW), lambda i: (i, 0))]
    if want_next:
        out_shapes.append(jax.ShapeDtypeStruct((EPAD, TW), jnp.float32))
        out_specs.append(pl.BlockSpec((BE, TW), lambda i: (i, 0)))

    return pl.pallas_call(
        body,
        grid=(nb,),
        in_specs=[
            pl.BlockSpec((BE, TW), lambda i: (i, 0)),
            pl.BlockSpec((BE, TW), lambda i: (i + nb, 0)),
            pl.BlockSpec((BE, TW), lambda i: (i, 0)),
            pl.BlockSpec((TW, 40), lambda i: (0, 0)),
            pl.BlockSpec((TW, 40), lambda i: (0, 0)),
            pl.BlockSpec((TW, 40), lambda i: (0, 0)),
            pl.BlockSpec((1, 40), lambda i: (0, 0)),
            pl.BlockSpec((40, TW), lambda i: (0, 0)),
            pl.BlockSpec((1, TW), lambda i: (0, 0)),
        ],
        out_specs=out_specs,
        out_shape=out_shapes,
        compiler_params=pltpu.CompilerParams(dimension_semantics=("parallel",)),
    )(hflat, hflat, e, W1s, W1d, W1e, b1, W2p, b2p)


def _tc_node_layer(htab, parts, Wh, Wa, b1, W2p, b2p):
    def body(h_ref, p_ref, Wh_ref, Wa_ref, b1_ref, W2_ref, b2_ref, o_ref):
        h = h_ref[...]
        agg = p_ref[0] + p_ref[1]
        hid = jnp.dot(h, Wh_ref[...], preferred_element_type=jnp.float32)
        hid = hid + jnp.dot(agg, Wa_ref[...], preferred_element_type=jnp.float32)
        hid = jnp.maximum(hid + b1_ref[...], 0.0)
        hn = jnp.dot(hid, W2_ref[...], preferred_element_type=jnp.float32) + b2_ref[...]
        o_ref[...] = ALPHA * h + (1.0 - ALPHA) * hn

    return pl.pallas_call(
        body,
        out_shape=jax.ShapeDtypeStruct((NPAD, TW), jnp.float32),
    )(htab, parts, Wh, Wa, b1, W2p, b2p)


def _tc_final(htab, parts, Wh, Wa, b1, W2p, b2p, Wb1, bb1, Wb2, bb2, Wh1, bh1, Wh2, bh2):
    def body(h_ref, p_ref, Wh_ref, Wa_ref, b1_ref, W2_ref, b2_ref,
             Wb1_ref, bb1_ref, Wb2_ref, bb2_ref, Wh1_ref, bh1_ref, Wh2_ref,
             bh2_ref, o_ref):
        h = h_ref[...]
        agg = p_ref[0] + p_ref[1]
        hid = jnp.dot(h, Wh_ref[...], preferred_element_type=jnp.float32)
        hid = hid + jnp.dot(agg, Wa_ref[...], preferred_element_type=jnp.float32)
        hid = jnp.maximum(hid + b1_ref[...], 0.0)
        hn = jnp.dot(hid, W2_ref[...], preferred_element_type=jnp.float32) + b2_ref[...]
        hf = (ALPHA * h + (1.0 - ALPHA) * hn)[:N_NODES]
        bhid = jnp.maximum(
            jnp.dot(hf, Wb1_ref[...], preferred_element_type=jnp.float32) + bb1_ref[...], 0.0)
        b8 = jnp.dot(bhid, Wb2_ref[...], preferred_element_type=jnp.float32) + bb2_ref[...]
        hhid = jnp.maximum(
            jnp.dot(hf, Wh1_ref[...], preferred_element_type=jnp.float32) + bh1_ref[...], 0.0)
        h8 = jnp.dot(hhid, Wh2_ref[...], preferred_element_type=jnp.float32) + bh2_ref[...]
        beta = jax.nn.sigmoid(b8[:, 0:1])
        o_ref[...] = jnp.concatenate([h8[:, 0:2], beta], axis=1)

    return pl.pallas_call(
        body,
        out_shape=jax.ShapeDtypeStruct((N_NODES, 3), jnp.float32),
    )(htab, parts, Wh, Wa, b1, W2p, b2p, Wb1, bb1, Wb2, bb2, Wh1, bh1, Wh2, bh2)


def _pad_rows(W, rows):
    out = jnp.zeros((rows, W.shape[1]), W.dtype)
    return out.at[: W.shape[0]].set(W)


def _pad_cols(W, cols):
    out = jnp.zeros((W.shape[0], cols), W.dtype)
    return out.at[:, : W.shape[1]].set(W)


def kernel(x, edge_index, edge_attr, enc_nW1, enc_nb1, enc_nW2, enc_nb2,
           enc_eW1, enc_eb1, enc_eW2, enc_eb2, relW1, relb1, relW2, relb2,
           objW1, objb1, objW2, objb2, betaW1, betab1, betaW2, betab2,
           hW1, hb1, hW2, hb2):
    f32 = jnp.float32
    i32 = jnp.int32
    src = edge_index[0]
    dst = edge_index[1]
    padi = jnp.full((EPAD - N_EDGES,), N_NODES, i32)
    src_p = jnp.concatenate([src, padi])
    dst_p = jnp.concatenate([dst, padi])
    gidx = jnp.concatenate([src_p, dst_p]).reshape(NW, NCH_G, CH)
    sidx = dst_p.reshape(NW, NCH_S, CH)
    xp = jnp.pad(x, ((0, NPAD - N_NODES), (0, 0)))
    eap = jnp.pad(edge_attr, ((0, EPAD - N_EDGES), (0, 0)))
    zeros_tab = jnp.zeros((NPAD, TW), f32)

    htab = _tc_node_encoder(
        xp, enc_nW1, enc_nb1.reshape(1, 40), _pad_cols(enc_nW2, TW),
        _pad_cols(enc_nb2.reshape(1, 5), TW))
    e = _tc_edge_encoder(
        eap, enc_eW1, enc_eb1.reshape(1, 40), _pad_cols(enc_eW2, TW),
        _pad_cols(enc_eb2.reshape(1, 4), TW))

    out = None
    for l in range(relW1.shape[0]):
        W1 = relW1[l]
        W1s = _pad_rows(W1[0:5], TW)
        W1d = _pad_rows(W1[5:10], TW)
        W1e = _pad_rows(W1[10:14], TW)
        b1 = relb1[l].reshape(1, 40)
        W2p = _pad_cols(relW2[l], TW)
        b2p = _pad_cols(relb2[l].reshape(1, 4), TW)

        gout = _sc_gather(htab, gidx)
        hflat = gout.reshape(2 * EPAD, TW)
        last = l == relW1.shape[0] - 1
        if last:
            (e_new,) = _tc_edge_layer(hflat, e, W1s, W1d, W1e, b1, W2p, b2p, False)
        else:
            e_new, e = _tc_edge_layer(hflat, e, W1s, W1d, W1e, b1, W2p, b2p, True)

        parts = _sc_scatter_add(e_new.reshape(NW, NCH_S, CH, TW), sidx, zeros_tab)

        oW1 = objW1[l]
        Wh = _pad_rows(oW1[0:5], TW)
        Wa = _pad_rows(oW1[5:9], TW)
        ob1 = objb1[l].reshape(1, 40)
        oW2p = _pad_cols(objW2[l], TW)
        ob2p = _pad_cols(objb2[l].reshape(1, 5), TW)
        if not last:
            htab = _tc_node_layer(htab, parts, Wh, Wa, ob1, oW2p, ob2p)
        else:
            out = _tc_final(
                htab, parts, Wh, Wa, ob1, oW2p, ob2p,
                _pad_rows(betaW1, TW), betab1.reshape(1, 40),
                _pad_cols(betaW2, TW), _pad_cols(betab2.reshape(1, 1), TW),
                _pad_rows(hW1, TW), hb1.reshape(1, 40),
                _pad_cols(hW2, TW), _pad_cols(hb2.reshape(1, 2), TW))
    return out


# speedup vs baseline: 2.9194x; 1.0816x over previous
import functools

import jax
import jax.numpy as jnp
from jax import lax
from jax.experimental import pallas as pl
from jax.experimental.pallas import tpu as pltpu
from jax.experimental.pallas import tpu_sc as plsc

ALPHA = 0.5
N_NODES = 10000
N_EDGES = 320000
NPAD = 10016
EPAD = 323584
NW = 32
CH = 128
NCH_S = 79
NCH_G = 158
TW = 8
BE = 2048
STRIPE = NPAD // 16


NBUF = 12
PREF = 6


def _sc_gather(table, idx):
    mesh = plsc.VectorSubcoreMesh(core_axis_name="c", subcore_axis_name="s")

    @functools.partial(
        pl.kernel,
        mesh=mesh,
        out_type=jax.ShapeDtypeStruct((NW, NCH_G, CH, TW), jnp.float32),
        compiler_params=pltpu.CompilerParams(use_tc_tiling_on_sc=False),
        scratch_types=[
            pltpu.VMEM((NCH_G, CH), jnp.int32),
            pltpu.VMEM((NBUF, CH, TW), jnp.float32),
            pltpu.SemaphoreType.DMA((NBUF,)),
            pltpu.SemaphoreType.DMA((NBUF,)),
        ],
    )
    def k(table_hbm, idx_hbm, out_hbm, idx_v, rows_v, gsem, osem):
        w = lax.axis_index("s") * 2 + lax.axis_index("c")
        pltpu.sync_copy(idx_hbm.at[w], idx_v)
        for b in range(PREF):
            pltpu.async_copy(table_hbm.at[idx_v.at[b]], rows_v.at[b], gsem.at[b])

        def body(j, carry):
            jn = j + PREF
            bn = jn % NBUF
            b = j % NBUF

            @pl.when(jn < NCH_G)
            def _():
                @pl.when(jn >= NBUF)
                def _():
                    pltpu.make_async_copy(rows_v.at[bn], out_hbm.at[w, 0],
                                          osem.at[bn]).wait()
                pltpu.async_copy(table_hbm.at[idx_v.at[jn]], rows_v.at[bn],
                                 gsem.at[bn])

            pltpu.make_async_copy(table_hbm.at[pl.ds(0, CH)], rows_v.at[b],
                                  gsem.at[b]).wait()
            pltpu.async_copy(rows_v.at[b], out_hbm.at[w, j], osem.at[b])
            return carry

        lax.fori_loop(0, NCH_G, body, 0)
        for b in range(NBUF):
            pltpu.make_async_copy(rows_v.at[b], out_hbm.at[w, 0],
                                  osem.at[b]).wait()

    return k(table, idx)


def _sc_scatter_add(vals, idx, zeros_tab):
    mesh = plsc.VectorSubcoreMesh(core_axis_name="c", subcore_axis_name="s")

    @functools.partial(
        pl.kernel,
        mesh=mesh,
        out_type=jax.ShapeDtypeStruct((2, NPAD, TW), jnp.float32),
        compiler_params=pltpu.CompilerParams(use_tc_tiling_on_sc=False),
        scratch_types=[
            pltpu.VMEM((NCH_S, CH), jnp.int32),
            pltpu.VMEM((NBUF, CH, TW), jnp.float32),
            pltpu.VMEM((STRIPE, TW), jnp.float32),
            pltpu.VMEM_SHARED((NPAD, TW), jnp.float32),
            pltpu.SemaphoreType.DMA((NBUF,)),
            pltpu.SemaphoreType.DMA((NBUF,)),
        ],
    )
    def k(vals_hbm, idx_hbm, z_hbm, out_hbm, idx_v, vals_v, stripe_v, acc_sh,
          vsem, ssem):
        c = lax.axis_index("c")
        s = lax.axis_index("s")
        w = s * 2 + c
        pltpu.sync_copy(z_hbm.at[pl.ds(s * STRIPE, STRIPE)], stripe_v)
        pltpu.sync_copy(stripe_v, acc_sh.at[pl.ds(s * STRIPE, STRIPE)])
        plsc.subcore_barrier()
        pltpu.sync_copy(idx_hbm.at[w], idx_v)
        for b in range(PREF):
            pltpu.async_copy(vals_hbm.at[w, b], vals_v.at[b], vsem.at[b])

        def body(j, carry):
            jn = j + PREF
            bn = jn % NBUF
            b = j % NBUF

            @pl.when(jn < NCH_S)
            def _():
                @pl.when(jn >= NBUF)
                def _():
                    pltpu.make_async_copy(vals_v.at[bn],
                                          acc_sh.at[pl.ds(0, CH)],
                                          ssem.at[bn]).wait()
                pltpu.async_copy(vals_hbm.at[w, jn], vals_v.at[bn], vsem.at[bn])

            pltpu.make_async_copy(vals_hbm.at[w, 0], vals_v.at[b],
                                  vsem.at[b]).wait()
            pltpu.async_copy(vals_v.at[b], acc_sh.at[idx_v.at[j]], ssem.at[b],
                             add=True)
            return carry

        lax.fori_loop(0, NCH_S, body, 0)
        for b in range(NBUF):
            pltpu.make_async_copy(vals_v.at[b], acc_sh.at[pl.ds(0, CH)],
                                  ssem.at[b]).wait()
        plsc.subcore_barrier()
        pltpu.sync_copy(acc_sh.at[pl.ds(s * STRIPE, STRIPE)], stripe_v)
        pltpu.sync_copy(stripe_v, out_hbm.at[c, pl.ds(s * STRIPE, STRIPE)])

    return k(vals, idx, zeros_tab)


def _tc_node_encoder(xp, W1, b1, W2p, b2p):
    def body(x_ref, W1_ref, b1_ref, W2_ref, b2_ref, o_ref):
        hid = jnp.dot(x_ref[...], W1_ref[...], preferred_element_type=jnp.float32)
        hid = jnp.maximum(hid + b1_ref[...], 0.0)
        o_ref[...] = jnp.dot(hid, W2_ref[...], preferred_element_type=jnp.float32) + b2_ref[...]

    return pl.pallas_call(
        body,
        out_shape=jax.ShapeDtypeStruct((NPAD, TW), jnp.float32),
    )(xp, W1, b1, W2p, b2p)


def _tc_edge_layer0(hflat, eap, eW1, eb1, eW2p, eb2p, W1cat, b1, W2p, b2p):
    nb = EPAD // BE

    def body(hs_ref, hd_ref, a_ref, eW1_ref, eb1_ref, eW2_ref, eb2_ref,
             W1_ref, b1_ref, W2_ref, b2_ref, en_ref, enext_ref):
        ehid = jnp.dot(a_ref[...], eW1_ref[...], preferred_element_type=jnp.float32)
        ehid = jnp.maximum(ehid + eb1_ref[...], 0.0)
        e0 = jnp.dot(ehid, eW2_ref[...], preferred_element_type=jnp.float32) + eb2_ref[...]
        hcat = jnp.concatenate([hs_ref[...], hd_ref[...], e0], axis=1)
        hid = jnp.dot(hcat, W1_ref[...], preferred_element_type=jnp.float32)
        hid = jnp.maximum(hid + b1_ref[...], 0.0)
        en = jnp.dot(hid, W2_ref[...], preferred_element_type=jnp.float32) + b2_ref[...]
        en_ref[...] = en
        enext_ref[...] = ALPHA * e0 + (1.0 - ALPHA) * en

    return pl.pallas_call(
        body,
        grid=(nb,),
        in_specs=[
            pl.BlockSpec((BE, TW), lambda i: (i, 0)),
            pl.BlockSpec((BE, TW), lambda i: (i + nb, 0)),
            pl.BlockSpec((BE, 16), lambda i: (i, 0)),
            pl.BlockSpec((16, 40), lambda i: (0, 0)),
            pl.BlockSpec((1, 40), lambda i: (0, 0)),
            pl.BlockSpec((40, TW), lambda i: (0, 0)),
            pl.BlockSpec((1, TW), lambda i: (0, 0)),
            pl.BlockSpec((3 * TW, 40), lambda i: (0, 0)),
            pl.BlockSpec((1, 40), lambda i: (0, 0)),
            pl.BlockSpec((40, TW), lambda i: (0, 0)),
            pl.BlockSpec((1, TW), lambda i: (0, 0)),
        ],
        out_specs=[
            pl.BlockSpec((BE, TW), lambda i: (i, 0)),
            pl.BlockSpec((BE, TW), lambda i: (i, 0)),
        ],
        out_shape=[
            jax.ShapeDtypeStruct((EPAD, TW), jnp.float32),
            jax.ShapeDtypeStruct((EPAD, TW), jnp.float32),
        ],
        compiler_params=pltpu.CompilerParams(dimension_semantics=("parallel",)),
    )(hflat, hflat, eap, eW1, eb1, eW2p, eb2p, W1cat, b1, W2p, b2p)


def _tc_edge_layer(hflat, e, W1cat, b1, W2p, b2p, want_next):
    nb = EPAD // BE

    def body(hs_ref, hd_ref, e_ref, W1_ref, b1_ref, W2_ref, b2_ref,
             en_ref, *rest):
        hcat = jnp.concatenate([hs_ref[...], hd_ref[...], e_ref[...]], axis=1)
        hid = jnp.dot(hcat, W1_ref[...], preferred_element_type=jnp.float32)
        hid = jnp.maximum(hid + b1_ref[...], 0.0)
        en = jnp.dot(hid, W2_ref[...], preferred_element_type=jnp.float32) + b2_ref[...]
        en_ref[...] = en
        if rest:
            rest[0][...] = ALPHA * e_ref[...] + (1.0 - ALPHA) * en

    out_shapes = [jax.ShapeDtypeStruct((EPAD, TW), jnp.float32)]
    out_specs = [pl.BlockSpec((BE, TW), lambda i: (i, 0))]
    if want_next:
        out_shapes.append(jax.ShapeDtypeStruct((EPAD, TW), jnp.float32))
        out_specs.append(pl.BlockSpec((BE, TW), lambda i: (i, 0)))

    return pl.pallas_call(
        body,
        grid=(nb,),
        in_specs=[
            pl.BlockSpec((BE, TW), lambda i: (i, 0)),
            pl.BlockSpec((BE, TW), lambda i: (i + nb, 0)),
            pl.BlockSpec((BE, TW), lambda i: (i, 0)),
            pl.BlockSpec((3 * TW, 40), lambda i: (0, 0)),
            pl.BlockSpec((1, 40), lambda i: (0, 0)),
            pl.BlockSpec((40, TW), lambda i: (0, 0)),
            pl.BlockSpec((1, TW), lambda i: (0, 0)),
        ],
        out_specs=out_specs,
        out_shape=out_shapes,
        compiler_params=pltpu.CompilerParams(dimension_semantics=("parallel",)),
    )(hflat, hflat, e, W1cat, b1, W2p, b2p)


def _tc_node_layer(htab, parts, Wcat, b1, W2p, b2p):
    def body(h_ref, p_ref, Wc_ref, b1_ref, W2_ref, b2_ref, o_ref):
        h = h_ref[...]
        agg = p_ref[0] + p_ref[1]
        hcat = jnp.concatenate([h, agg], axis=1)
        hid = jnp.dot(hcat, Wc_ref[...], preferred_element_type=jnp.float32)
        hid = jnp.maximum(hid + b1_ref[...], 0.0)
        hn = jnp.dot(hid, W2_ref[...], preferred_element_type=jnp.float32) + b2_ref[...]
        o_ref[...] = ALPHA * h + (1.0 - ALPHA) * hn

    return pl.pallas_call(
        body,
        out_shape=jax.ShapeDtypeStruct((NPAD, TW), jnp.float32),
    )(htab, parts, Wcat, b1, W2p, b2p)


def _tc_final(htab, parts, Wcat, b1, W2p, b2p, Wb1, bb1, Wb2, bb2, Wh1, bh1, Wh2, bh2):
    def body(h_ref, p_ref, Wc_ref, b1_ref, W2_ref, b2_ref,
             Wb1_ref, bb1_ref, Wb2_ref, bb2_ref, Wh1_ref, bh1_ref, Wh2_ref,
             bh2_ref, o_ref):
        h = h_ref[...]
        agg = p_ref[0] + p_ref[1]
        hcat = jnp.concatenate([h, agg], axis=1)
        hid = jnp.dot(hcat, Wc_ref[...], preferred_element_type=jnp.float32)
        hid = jnp.maximum(hid + b1_ref[...], 0.0)
        hn = jnp.dot(hid, W2_ref[...], preferred_element_type=jnp.float32) + b2_ref[...]
        hf = (ALPHA * h + (1.0 - ALPHA) * hn)[:N_NODES]
        bhid = jnp.maximum(
            jnp.dot(hf, Wb1_ref[...], preferred_element_type=jnp.float32) + bb1_ref[...], 0.0)
        b8 = jnp.dot(bhid, Wb2_ref[...], preferred_element_type=jnp.float32) + bb2_ref[...]
        hhid = jnp.maximum(
            jnp.dot(hf, Wh1_ref[...], preferred_element_type=jnp.float32) + bh1_ref[...], 0.0)
        h8 = jnp.dot(hhid, Wh2_ref[...], preferred_element_type=jnp.float32) + bh2_ref[...]
        beta = jax.nn.sigmoid(b8[:, 0:1])
        o_ref[...] = jnp.concatenate([h8[:, 0:2], beta], axis=1)

    return pl.pallas_call(
        body,
        out_shape=jax.ShapeDtypeStruct((N_NODES, 3), jnp.float32),
    )(htab, parts, Wcat, b1, W2p, b2p, Wb1, bb1, Wb2, bb2, Wh1, bh1, Wh2, bh2)


def _pad_rows(W, rows):
    out = jnp.zeros((rows, W.shape[1]), W.dtype)
    return out.at[: W.shape[0]].set(W)


def _pad_cols(W, cols):
    out = jnp.zeros((W.shape[0], cols), W.dtype)
    return out.at[:, : W.shape[1]].set(W)


def kernel(x, edge_index, edge_attr, enc_nW1, enc_nb1, enc_nW2, enc_nb2,
           enc_eW1, enc_eb1, enc_eW2, enc_eb2, relW1, relb1, relW2, relb2,
           objW1, objb1, objW2, objb2, betaW1, betab1, betaW2, betab2,
           hW1, hb1, hW2, hb2):
    f32 = jnp.float32
    i32 = jnp.int32
    src = edge_index[0]
    dst = edge_index[1]
    padi = jnp.full((EPAD - N_EDGES,), N_NODES, i32)
    src_p = jnp.concatenate([src, padi])
    dst_p = jnp.concatenate([dst, padi])
    gidx = jnp.concatenate([src_p, dst_p]).reshape(NW, NCH_G, CH)
    sidx = dst_p.reshape(NW, NCH_S, CH)
    xp = jnp.pad(x, ((0, NPAD - N_NODES), (0, 0)))
    eap = jnp.pad(edge_attr, ((0, EPAD - N_EDGES), (0, 0)))
    zeros_tab = jnp.zeros((NPAD, TW), f32)

    htab = _tc_node_encoder(
        xp, enc_nW1, enc_nb1.reshape(1, 40), _pad_cols(enc_nW2, TW),
        _pad_cols(enc_nb2.reshape(1, 5), TW))

    out = None
    e = None
    for l in range(relW1.shape[0]):
        W1 = relW1[l]
        W1cat = jnp.concatenate([
            _pad_rows(W1[0:5], TW), _pad_rows(W1[5:10], TW),
            _pad_rows(W1[10:14], TW)], axis=0)
        b1 = relb1[l].reshape(1, 40)
        W2p = _pad_cols(relW2[l], TW)
        b2p = _pad_cols(relb2[l].reshape(1, 4), TW)

        gout = _sc_gather(htab, gidx)
        hflat = gout.reshape(2 * EPAD, TW)
        last = l == relW1.shape[0] - 1
        if l == 0:
            e_new, e = _tc_edge_layer0(
                hflat, eap, enc_eW1, enc_eb1.reshape(1, 40),
                _pad_cols(enc_eW2, TW), _pad_cols(enc_eb2.reshape(1, 4), TW),
                W1cat, b1, W2p, b2p)
        elif last:
            (e_new,) = _tc_edge_layer(hflat, e, W1cat, b1, W2p, b2p, False)
        else:
            e_new, e = _tc_edge_layer(hflat, e, W1cat, b1, W2p, b2p, True)

        parts = _sc_scatter_add(e_new.reshape(NW, NCH_S, CH, TW), sidx, zeros_tab)

        oW1 = objW1[l]
        Wcat = jnp.concatenate(
            [_pad_rows(oW1[0:5], TW), _pad_rows(oW1[5:9], TW)], axis=0)
        ob1 = objb1[l].reshape(1, 40)
        oW2p = _pad_cols(objW2[l], TW)
        ob2p = _pad_cols(objb2[l].reshape(1, 5), TW)
        if not last:
            htab = _tc_node_layer(htab, parts, Wcat, ob1, oW2p, ob2p)
        else:
            out = _tc_final(
                htab, parts, Wcat, ob1, oW2p, ob2p,
                _pad_rows(betaW1, TW), betab1.reshape(1, 40),
                _pad_cols(betaW2, TW), _pad_cols(betab2.reshape(1, 1), TW),
                _pad_rows(hW1, TW), hb1.reshape(1, 40),
                _pad_cols(hW2, TW), _pad_cols(hb2.reshape(1, 2), TW))
    return out
